# baseline (speedup 1.0000x reference)
import jax
import jax.numpy as jnp
from jax.experimental import pallas as pl
from jax.experimental.pallas import tpu as pltpu

_MEM_ROWS = 100000
_MEM_UNIT = 256
_D_OUT = 768
_BLK = 2000
_NBLK = _MEM_ROWS // _BLK
_MIN_SIM = 0.5


def _matvec_body(x_ref, w_ref, b_ref, o_ref):
    o_ref[...] = jax.lax.dot_general(
        x_ref[...], w_ref[...], (((1,), (1,)), ((), ())),
        preferred_element_type=jnp.float32,
        precision=jax.lax.Precision.DEFAULT) + b_ref[...]


def _scan_body(sv_ref, hp_ref, m_ref, mem_ref, memo_ref, head_ref,
               bs_ref, bi_ref):
    i = pl.program_id(0)
    hp = hp_ref[0]
    w = sv_ref[2]
    blk = mem_ref[...]
    rows = jax.lax.broadcasted_iota(jnp.int32, (_BLK, 1), 0) + i * _BLK
    over = (rows == hp) & (w > 0.5)
    upd = jnp.where(over, m_ref[...], blk)
    memo_ref[...] = upd
    d = upd - m_ref[...]
    d2 = jnp.sum(d * d, axis=1, keepdims=True)
    sims = 1.0 - jnp.sqrt(d2) / _MEM_UNIT
    bmax = jnp.max(sims)
    barg = jnp.min(jnp.where(sims == bmax, rows, jnp.int32(0x7FFFFFFF)))

    @pl.when(i == 0)
    def _init():
        bs_ref[0] = -jnp.inf
        bi_ref[0] = 0

    @pl.when(bmax > bs_ref[0])
    def _update():
        bs_ref[0] = bmax
        bi_ref[0] = barg

    @pl.when(i == _NBLK - 1)
    def _finish():
        s = sv_ref[0]
        j = sv_ref[1]
        jumped = jnp.where(bs_ref[0] > _MIN_SIM, bi_ref[0], 0)
        head0 = jnp.where(j > 0.5, jumped, hp)
        shift = (s * 3.0 - 1e-9).astype(jnp.int32) - 1
        head_ref[0] = jnp.mod(head0 + shift, _MEM_ROWS)


def _gather_body(hp_ref, memo_ref, o_ref):
    del hp_ref
    o_ref[...] = memo_ref[...]


def kernel(x, prev_read, mem, W, b, head_pos):
    xj = jnp.concatenate([x, prev_read], axis=0)[None, :]
    out = pl.pallas_call(
        _matvec_body,
        out_shape=jax.ShapeDtypeStruct((1, W.shape[0]), jnp.float32),
    )(xj, W, b[None, :])[0]
    y = out[:_D_OUT]
    sv = out[_D_OUT:_D_OUT + 3]
    m = out[_D_OUT + 3:]
    hp = jnp.asarray(head_pos, jnp.int32).reshape(1)

    mem_out, head = pl.pallas_call(
        _scan_body,
        grid=(_NBLK,),
        in_specs=[
            pl.BlockSpec(memory_space=pltpu.SMEM),
            pl.BlockSpec(memory_space=pltpu.SMEM),
            pl.BlockSpec((1, _MEM_UNIT), lambda i: (0, 0)),
            pl.BlockSpec((_BLK, _MEM_UNIT), lambda i: (i, 0)),
        ],
        out_specs=[
            pl.BlockSpec((_BLK, _MEM_UNIT), lambda i: (i, 0)),
            pl.BlockSpec(memory_space=pltpu.SMEM),
        ],
        out_shape=[
            jax.ShapeDtypeStruct((_MEM_ROWS, _MEM_UNIT), jnp.float32),
            jax.ShapeDtypeStruct((1,), jnp.int32),
        ],
        scratch_shapes=[pltpu.SMEM((1,), jnp.float32),
                        pltpu.SMEM((1,), jnp.int32)],
    )(sv, hp, m[None, :], mem)

    mem3d = mem_out.reshape(_MEM_ROWS, 1, _MEM_UNIT)
    new_read = pl.pallas_call(
        _gather_body,
        grid_spec=pltpu.PrefetchScalarGridSpec(
            num_scalar_prefetch=1,
            grid=(1,),
            in_specs=[pl.BlockSpec((1, 1, _MEM_UNIT), lambda i, h: (h[0], 0, 0))],
            out_specs=pl.BlockSpec((1, 1, _MEM_UNIT), lambda i, h: (0, 0, 0)),
        ),
        out_shape=jax.ShapeDtypeStruct((1, 1, _MEM_UNIT), jnp.float32),
    )(head, mem3d).reshape(_MEM_UNIT)

    return (y, new_read, mem_out)
